# baseline (speedup 1.0000x reference)
import functools

import jax
import jax.numpy as jnp
from jax import lax
from jax.experimental import pallas as pl
from jax.experimental.pallas import tpu as pltpu
from jax.experimental.pallas import tpu_sc as plsc

N = 10000
D = 128
E = 320000

NC = 2
NS = 16
NW = NC * NS

K = 128
C = 80
H = C // 2
EPW = C * K
E_PAD = NW * EPW
NPAD = 10240
NJUNK = NPAD - N
ZROWS = NPAD // NS
PADW = EPW - E // NW
DEGPAD = 10240
DZROWS = DEGPAD // NS

_MESH = plsc.VectorSubcoreMesh(core_axis_name="c", subcore_axis_name="s")


@functools.partial(
    pl.kernel,
    out_type=jax.ShapeDtypeStruct((NC, DEGPAD), jnp.float32),
    mesh=_MESH,
    scratch_types=[
        pltpu.VMEM((C, K), jnp.int32),
        pltpu.VMEM((K,), jnp.float32),
        pltpu.VMEM_SHARED((DEGPAD,), jnp.float32),
    ],
)
def _deg_kernel(dst_hbm, zeros_hbm, ones_hbm, deg_hbm, dst_v, ones_v, acc):
    c = lax.axis_index("c")
    s = lax.axis_index("s")
    pltpu.sync_copy(zeros_hbm, acc.at[pl.ds(s * DZROWS, DZROWS)])
    pltpu.sync_copy(dst_hbm.at[c, s], dst_v)
    pltpu.sync_copy(ones_hbm, ones_v)
    plsc.subcore_barrier()

    def chunk(j, carry):
        pltpu.sync_copy(ones_v, acc.at[dst_v.at[j]], add=True)
        return carry

    lax.fori_loop(0, C, chunk, 0)
    plsc.subcore_barrier()
    pltpu.sync_copy(acc.at[pl.ds(s * DZROWS, DZROWS)],
                    deg_hbm.at[c, pl.ds(s * DZROWS, DZROWS)])


@functools.partial(
    pl.kernel,
    out_type=jax.ShapeDtypeStruct((NC, NPAD, D), jnp.float32),
    mesh=_MESH,
    scratch_types=[
        pltpu.VMEM((H, K), jnp.int32),
        pltpu.VMEM((H, K), jnp.int32),
        pltpu.VMEM((2, K, D), jnp.float32),
        pltpu.SemaphoreType.DMA,
        pltpu.SemaphoreType.DMA,
        pltpu.SemaphoreType.DMA,
        pltpu.SemaphoreType.DMA,
        pltpu.VMEM_SHARED((NPAD, D), jnp.float32),
    ],
)
def _agg_kernel(g_hbm, src_hbm, dst_hbm, zrows_hbm, out_hbm,
                src_v, dst_v, rows_v, g0, g1, s0, s1, acc):
    c = lax.axis_index("c")
    s = lax.axis_index("s")
    pltpu.sync_copy(zrows_hbm, acc.at[pl.ds(s * ZROWS, ZROWS)])
    plsc.subcore_barrier()

    for h in range(2):
        pltpu.sync_copy(src_hbm.at[c, s, pl.ds(h * H, H)], src_v)
        pltpu.sync_copy(dst_hbm.at[c, s, pl.ds(h * H, H)], dst_v)
        pltpu.async_copy(g_hbm.at[src_v.at[0]], rows_v.at[0], g0)
        pltpu.async_copy(g_hbm.at[src_v.at[1]], rows_v.at[1], g1)

        def pair(i, carry):
            j = 2 * i
            pltpu.make_async_copy(g_hbm.at[src_v.at[0]], rows_v.at[0], g0).wait()
            pltpu.async_copy(rows_v.at[0], acc.at[dst_v.at[j]], s0, add=True)
            pltpu.make_async_copy(g_hbm.at[src_v.at[1]], rows_v.at[1], g1).wait()
            pltpu.async_copy(rows_v.at[1], acc.at[dst_v.at[j + 1]], s1, add=True)
            pltpu.make_async_copy(rows_v.at[0], acc.at[dst_v.at[0]], s0).wait()

            @pl.when(j + 2 < H)
            def _():
                pltpu.async_copy(g_hbm.at[src_v.at[j + 2]], rows_v.at[0], g0)

            pltpu.make_async_copy(rows_v.at[1], acc.at[dst_v.at[0]], s1).wait()

            @pl.when(j + 3 < H)
            def _():
                pltpu.async_copy(g_hbm.at[src_v.at[j + 3]], rows_v.at[1], g1)

            return carry

        lax.fori_loop(0, H // 2, pair, 0)
    plsc.subcore_barrier()
    pltpu.sync_copy(acc.at[pl.ds(s * ZROWS, ZROWS)],
                    out_hbm.at[c, pl.ds(s * ZROWS, ZROWS)])


_BM = 2000


def _mm1_body(x_ref, w_ref, d_ref, g_ref, dis_ref):
    deg = d_ref[0] + d_ref[1] + 1.0
    dis = lax.rsqrt(deg)
    h = jnp.dot(x_ref[...], w_ref[...], preferred_element_type=jnp.float32)
    g_ref[...] = h * dis
    dis_ref[...] = dis


def _mm2_body(s_ref, g_ref, dis_ref, b_ref, w_ref, g2_ref):
    dis = dis_ref[...]
    a = (s_ref[0] + s_ref[1] + g_ref[...]) * dis + b_ref[...]
    a = jnp.maximum(a, 0.0)
    h = jnp.dot(a, w_ref[...], preferred_element_type=jnp.float32)
    g2_ref[...] = h * dis


def _out_body(s_ref, g_ref, dis_ref, b_ref, o_ref):
    o_ref[...] = (s_ref[0] + s_ref[1] + g_ref[...]) * dis_ref[...] + b_ref[...]


_GRID = (N // _BM,)
_row_blk = pl.BlockSpec((_BM, D), lambda i: (i, 0))
_s_blk = pl.BlockSpec((NC, _BM, D), lambda i: (0, i, 0))
_d_blk = pl.BlockSpec((NC, _BM, 1), lambda i: (0, i, 0))
_dis_blk = pl.BlockSpec((_BM, 1), lambda i: (i, 0))
_w_blk = pl.BlockSpec((D, D), lambda i: (0, 0))
_b_blk = pl.BlockSpec((1, D), lambda i: (0, 0))

_mm1 = pl.pallas_call(
    _mm1_body,
    grid=_GRID,
    in_specs=[_row_blk, _w_blk, _d_blk],
    out_specs=[_row_blk, _dis_blk],
    out_shape=[
        jax.ShapeDtypeStruct((N, D), jnp.float32),
        jax.ShapeDtypeStruct((N, 1), jnp.float32),
    ],
)

_mm2 = pl.pallas_call(
    _mm2_body,
    grid=_GRID,
    in_specs=[_s_blk, _row_blk, _dis_blk, _b_blk, _w_blk],
    out_specs=_row_blk,
    out_shape=jax.ShapeDtypeStruct((N, D), jnp.float32),
)

_out_tc = pl.pallas_call(
    _out_body,
    grid=_GRID,
    in_specs=[_s_blk, _row_blk, _dis_blk, _b_blk],
    out_specs=_row_blk,
    out_shape=jax.ShapeDtypeStruct((N, D), jnp.float32),
)


def kernel(x, edge_index, W1, b1, W2, b2):
    src = edge_index[0].astype(jnp.int32).reshape(NW, E // NW)
    dst = edge_index[1].astype(jnp.int32).reshape(NW, E // NW)
    psrc = jnp.broadcast_to(jnp.arange(PADW, dtype=jnp.int32), (NW, PADW))
    pdst = jnp.broadcast_to(N + jnp.arange(PADW, dtype=jnp.int32) % NJUNK,
                            (NW, PADW))
    src_p = jnp.concatenate([src, psrc], axis=1).reshape(NC, NS, C, K)
    dst_p = jnp.concatenate([dst, pdst], axis=1).reshape(NC, NS, C, K)

    zeros1 = jnp.zeros((DZROWS,), jnp.float32)
    ones_k = jnp.ones((K,), jnp.float32)
    zrows = jnp.zeros((ZROWS, D), jnp.float32)

    deg2 = _deg_kernel(dst_p, zeros1, ones_k)
    d3 = deg2[:, :N].reshape(NC, N, 1)

    g1, dis = _mm1(x, W1, d3)
    s1 = _agg_kernel(g1, src_p, dst_p, zrows)
    g2 = _mm2(s1, g1, dis, b1.reshape(1, D), W2)
    s2 = _agg_kernel(g2, src_p, dst_p, zrows)
    out = _out_tc(s2, g2, dis, b2.reshape(1, D))
    return out


# speedup vs baseline: 1.0961x; 1.0961x over previous
import functools

import jax
import jax.numpy as jnp
from jax import lax
from jax.experimental import pallas as pl
from jax.experimental.pallas import tpu as pltpu
from jax.experimental.pallas import tpu_sc as plsc

N = 10000
D = 128
E = 320000

NC = 2
NS = 16
NW = NC * NS

K = 128
C = 80
H = C // 2
EPW = C * K
E_PAD = NW * EPW
NPAD = 10240
NJUNK = NPAD - N
ZROWS = NPAD // NS
PADW = EPW - E // NW
DEGPAD = 10240
DZROWS = DEGPAD // NS

_MESH = plsc.VectorSubcoreMesh(core_axis_name="c", subcore_axis_name="s")


@functools.partial(
    pl.kernel,
    out_type=jax.ShapeDtypeStruct((NC, DEGPAD), jnp.float32),
    mesh=_MESH,
    scratch_types=[
        pltpu.VMEM((C, K), jnp.int32),
        pltpu.VMEM((K,), jnp.float32),
        pltpu.VMEM_SHARED((DEGPAD,), jnp.float32),
    ],
)
def _deg_kernel(dst_hbm, zeros_hbm, ones_hbm, deg_hbm, dst_v, ones_v, acc):
    c = lax.axis_index("c")
    s = lax.axis_index("s")
    pltpu.sync_copy(zeros_hbm, acc.at[pl.ds(s * DZROWS, DZROWS)])
    pltpu.sync_copy(dst_hbm.at[c, s], dst_v)
    pltpu.sync_copy(ones_hbm, ones_v)
    plsc.subcore_barrier()

    def chunk(j, carry):
        pltpu.sync_copy(ones_v, acc.at[dst_v.at[j]], add=True)
        return carry

    lax.fori_loop(0, C, chunk, 0)
    plsc.subcore_barrier()
    pltpu.sync_copy(acc.at[pl.ds(s * DZROWS, DZROWS)],
                    deg_hbm.at[c, pl.ds(s * DZROWS, DZROWS)])


@functools.partial(
    pl.kernel,
    out_type=jax.ShapeDtypeStruct((NC, NPAD, D), jnp.float32),
    mesh=_MESH,
    scratch_types=[
        pltpu.VMEM((H, K), jnp.int32),
        pltpu.VMEM((H, K), jnp.int32),
        pltpu.VMEM((2, K, D), jnp.float32),
        pltpu.SemaphoreType.DMA,
        pltpu.SemaphoreType.DMA,
        pltpu.SemaphoreType.DMA,
        pltpu.VMEM_SHARED((NPAD, D), jnp.float32),
    ],
)
def _agg_kernel(g_hbm, src_hbm, dst_hbm, zrows_hbm, out_hbm,
                src_v, dst_v, rows_v, g0, g1, zs, acc):
    c = lax.axis_index("c")
    s = lax.axis_index("s")
    pltpu.async_copy(zrows_hbm, acc.at[pl.ds(s * ZROWS, ZROWS)], zs)
    pltpu.sync_copy(src_hbm.at[c, s, pl.ds(0, H)], src_v)
    pltpu.sync_copy(dst_hbm.at[c, s, pl.ds(0, H)], dst_v)
    pltpu.async_copy(g_hbm.at[src_v.at[0]], rows_v.at[0], g0)
    pltpu.make_async_copy(zrows_hbm, acc.at[pl.ds(0, ZROWS)], zs).wait()
    plsc.subcore_barrier()

    for h in range(2):
        if h:
            pltpu.sync_copy(src_hbm.at[c, s, pl.ds(h * H, H)], src_v)
            pltpu.sync_copy(dst_hbm.at[c, s, pl.ds(h * H, H)], dst_v)
            pltpu.async_copy(g_hbm.at[src_v.at[0]], rows_v.at[0], g0)

        def pair(i, carry):
            j = 2 * i
            pltpu.make_async_copy(g_hbm.at[src_v.at[0]], rows_v.at[0], g0).wait()
            pltpu.async_copy(g_hbm.at[src_v.at[j + 1]], rows_v.at[1], g1)
            pltpu.sync_copy(rows_v.at[0], acc.at[dst_v.at[j]], add=True)
            pltpu.make_async_copy(g_hbm.at[src_v.at[1]], rows_v.at[1], g1).wait()

            @pl.when(j + 2 < H)
            def _():
                pltpu.async_copy(g_hbm.at[src_v.at[j + 2]], rows_v.at[0], g0)

            pltpu.sync_copy(rows_v.at[1], acc.at[dst_v.at[j + 1]], add=True)
            return carry

        lax.fori_loop(0, H // 2, pair, 0)
    plsc.subcore_barrier()
    pltpu.sync_copy(acc.at[pl.ds(s * ZROWS, ZROWS)],
                    out_hbm.at[c, pl.ds(s * ZROWS, ZROWS)])


_BM = 2000


def _mm1_body(x_ref, w_ref, d_ref, g_ref, dis_ref):
    deg = d_ref[0] + d_ref[1] + 1.0
    dis = lax.rsqrt(deg)
    h = jnp.dot(x_ref[...], w_ref[...], preferred_element_type=jnp.float32)
    g_ref[...] = h * dis
    dis_ref[...] = dis


def _mm2_body(s_ref, g_ref, dis_ref, b_ref, w_ref, g2_ref):
    dis = dis_ref[...]
    a = (s_ref[0] + s_ref[1] + g_ref[...]) * dis + b_ref[...]
    a = jnp.maximum(a, 0.0)
    h = jnp.dot(a, w_ref[...], preferred_element_type=jnp.float32)
    g2_ref[...] = h * dis


def _out_body(s_ref, g_ref, dis_ref, b_ref, o_ref):
    o_ref[...] = (s_ref[0] + s_ref[1] + g_ref[...]) * dis_ref[...] + b_ref[...]


_GRID = (N // _BM,)
_row_blk = pl.BlockSpec((_BM, D), lambda i: (i, 0))
_s_blk = pl.BlockSpec((NC, _BM, D), lambda i: (0, i, 0))
_d_blk = pl.BlockSpec((NC, _BM, 1), lambda i: (0, i, 0))
_dis_blk = pl.BlockSpec((_BM, 1), lambda i: (i, 0))
_w_blk = pl.BlockSpec((D, D), lambda i: (0, 0))
_b_blk = pl.BlockSpec((1, D), lambda i: (0, 0))

_mm1 = pl.pallas_call(
    _mm1_body,
    grid=_GRID,
    in_specs=[_row_blk, _w_blk, _d_blk],
    out_specs=[_row_blk, _dis_blk],
    out_shape=[
        jax.ShapeDtypeStruct((N, D), jnp.float32),
        jax.ShapeDtypeStruct((N, 1), jnp.float32),
    ],
)

_mm2 = pl.pallas_call(
    _mm2_body,
    grid=_GRID,
    in_specs=[_s_blk, _row_blk, _dis_blk, _b_blk, _w_blk],
    out_specs=_row_blk,
    out_shape=jax.ShapeDtypeStruct((N, D), jnp.float32),
)

_out_tc = pl.pallas_call(
    _out_body,
    grid=_GRID,
    in_specs=[_s_blk, _row_blk, _dis_blk, _b_blk],
    out_specs=_row_blk,
    out_shape=jax.ShapeDtypeStruct((N, D), jnp.float32),
)


def kernel(x, edge_index, W1, b1, W2, b2):
    src = edge_index[0].astype(jnp.int32).reshape(NW, E // NW)
    dst = edge_index[1].astype(jnp.int32).reshape(NW, E // NW)
    psrc = jnp.broadcast_to(jnp.arange(PADW, dtype=jnp.int32), (NW, PADW))
    pdst = jnp.broadcast_to(N + jnp.arange(PADW, dtype=jnp.int32) % NJUNK,
                            (NW, PADW))
    src_p = jnp.concatenate([src, psrc], axis=1).reshape(NC, NS, C, K)
    dst_p = jnp.concatenate([dst, pdst], axis=1).reshape(NC, NS, C, K)

    zeros1 = jnp.zeros((DZROWS,), jnp.float32)
    ones_k = jnp.ones((K,), jnp.float32)
    zrows = jnp.zeros((ZROWS, D), jnp.float32)

    deg2 = _deg_kernel(dst_p, zeros1, ones_k)
    d3 = deg2.reshape(NC, DEGPAD, 1)

    g1, dis = _mm1(x, W1, d3)
    s1 = _agg_kernel(g1, src_p, dst_p, zrows)
    g2 = _mm2(s1, g1, dis, b1.reshape(1, D), W2)
    s2 = _agg_kernel(g2, src_p, dst_p, zrows)
    out = _out_tc(s2, g2, dis, b2.reshape(1, D))
    return out
